# baseline (speedup 1.0000x reference)
import functools

import jax
import jax.numpy as jnp
from jax import lax
from jax.experimental import pallas as pl
from jax.experimental.pallas import tpu as pltpu
from jax.experimental.pallas import tpu_sc as plsc

_NC = 2
_NS = 16
_NW = _NC * _NS
_LANES = 16


def _sc_pool(trig_idx_t, rate_idx, trig_bf, rate_bf, B, S, T, E, L):
    assert B % (2 * _NW) == 0
    spw = B // _NW
    tph = T // 2
    rph = tph * S
    ch = 80
    assert rph % ch == 0 and ch % 8 == 0
    nch = rph // ch
    ej = E // _LANES
    rchunks = []
    off = 0
    while off < L:
        n = min(128, L - off)
        if L - off > 128:
            n -= n % 8
        rchunks.append((off, n))
        off += n

    mesh = plsc.VectorSubcoreMesh(core_axis_name="c", subcore_axis_name="s")

    @functools.partial(
        pl.kernel,
        out_type=(
            jax.ShapeDtypeStruct((B * T, E), jnp.float32),
            jax.ShapeDtypeStruct((B, E), jnp.float32),
        ),
        mesh=mesh,
        compiler_params=pltpu.CompilerParams(use_tc_tiling_on_sc=False,
                                             needs_layout_passes=False),
        scratch_types=[
            pltpu.VMEM((2, rph), jnp.int32),
            pltpu.VMEM((2, L), jnp.int32),
            pltpu.VMEM((2, rph, E), jnp.bfloat16),
            pltpu.VMEM((2, L, E), jnp.bfloat16),
            pltpu.VMEM((T, E), jnp.float32),
            pltpu.VMEM((1, E), jnp.float32),
            pltpu.SemaphoreType.DMA,
            pltpu.SemaphoreType.DMA,
            pltpu.SemaphoreType.DMA,
            pltpu.SemaphoreType.DMA,
        ],
    )
    def pool(ti_hbm, ri_hbm, tt_hbm, rt_hbm, tout_hbm, rout_hbm,
             idx_v, ridx_v, buf, rbuf, featv, ratev, gsem0, gsem1, rsem0,
             rsem1):
        wid = lax.axis_index("s") * _NC + lax.axis_index("c")
        base_b = wid * spw
        gsems = (gsem0, gsem1)
        rsems = (rsem0, rsem1)
        rps = T * S

        def fire_half(i, half, hb):
            start = (base_b + i) * rps + half * rph
            pltpu.sync_copy(ti_hbm.at[pl.ds(start, rph)], idx_v.at[hb])
            for k in range(nch):
                pltpu.async_copy(
                    tt_hbm.at[idx_v.at[hb, pl.ds(k * ch, ch)]],
                    buf.at[hb, pl.ds(k * ch, ch)], gsems[hb])

        def wait_half(hb):
            pltpu.make_async_copy(
                tt_hbm.at[pl.ds(0, rph)], buf.at[hb], gsems[hb]).wait()

        def fire_rate(i, rb):
            start = (base_b + i) * L
            pltpu.sync_copy(ri_hbm.at[pl.ds(start, L)], ridx_v.at[rb])
            for (o, n) in rchunks:
                pltpu.async_copy(
                    rt_hbm.at[ridx_v.at[rb, pl.ds(o, n)]],
                    rbuf.at[rb, pl.ds(o, n)], rsems[rb])

        def wait_rate(rb):
            pltpu.make_async_copy(
                rt_hbm.at[pl.ds(0, L)], rbuf.at[rb], rsems[rb]).wait()

        def row_terms(ref, *ix):
            terms = []
            for g in range(E // 32):
                packed = ref[(*ix, pl.ds(g * 32, 32))]
                a, b = plsc.unpack(packed,
                                   format=plsc.PackFormat.INTERLEAVED)
                terms.extend((a, b))
            return terms

        def accum_half(half, hb):
            def tbody(tt, c):
                accs = [jnp.zeros((_LANES,), jnp.float32) for _ in range(ej)]
                for s in range(S):
                    terms = row_terms(buf, hb, tt * S + s)
                    for j in range(ej):
                        accs[j] = accs[j] + terms[j]
                for j in range(ej):
                    featv[half * tph + tt, pl.ds(j * _LANES, _LANES)] = (
                        accs[j] * (1.0 / S))
                return c

            lax.fori_loop(0, tph, tbody, 0)

        def accum_rate(rb):
            def rbody(s, accs):
                terms = row_terms(rbuf, rb, s)
                return tuple(accs[j] + terms[j] for j in range(ej))

            raccs = lax.fori_loop(
                0, L, rbody,
                tuple(jnp.zeros((_LANES,), jnp.float32) for _ in range(ej)))
            for j in range(ej):
                ratev[0, pl.ds(j * _LANES, _LANES)] = raccs[j] * (1.0 / L)

        fire_half(0, 0, 0)
        fire_rate(0, 0)

        def pair_body(g, carry):
            for p in range(2):
                i = g * 2 + p
                b = base_b + i
                fire_half(i, 1, 1)
                nxt = jnp.minimum(i + 1, spw - 1)
                fire_rate(nxt, 1 - p)
                wait_half(0)
                accum_half(0, 0)
                fire_half(nxt, 0, 0)
                wait_half(1)
                accum_half(1, 1)
                pltpu.sync_copy(featv, tout_hbm.at[pl.ds(b * T, T)])
                wait_rate(p)
                accum_rate(p)
                pltpu.sync_copy(ratev, rout_hbm.at[pl.ds(b, 1)])
            return carry

        lax.fori_loop(0, spw // 2, pair_body, 0)
        wait_half(0)
        wait_rate(0)

    return pool(trig_idx_t, rate_idx, trig_bf, rate_bf)


def _mlp(trig_feat, rate_feat, W1p, b1, W2a, W2b, b2, W3, b3, B, T, E, H, C):
    blk = 256
    assert B % blk == 0

    def body(tf_ref, rf_ref, w1_ref, b1_ref, w2a_ref, w2b_ref, b2_ref,
             w3_ref, b3_ref, o_ref):
        x = tf_ref[...]
        h1 = jnp.dot(x, w1_ref[...], preferred_element_type=jnp.float32)
        h1 = jnp.maximum(h1 + b1_ref[...], 0.0)
        h2 = (jnp.dot(rf_ref[...], w2a_ref[...],
                      preferred_element_type=jnp.float32)
              + jnp.dot(h1, w2b_ref[...], preferred_element_type=jnp.float32))
        h2 = jnp.maximum(h2 + b2_ref[...], 0.0)
        o_ref[...] = (jnp.dot(h2, w3_ref[...],
                              preferred_element_type=jnp.float32)
                      + b3_ref[...])

    grid = (B // blk,)
    full = lambda shape: pl.BlockSpec(shape, lambda i: (0,) * len(shape))
    return pl.pallas_call(
        body,
        grid=grid,
        in_specs=[
            pl.BlockSpec((blk, T * E), lambda i: (i, 0)),
            pl.BlockSpec((blk, E), lambda i: (i, 0)),
            full((T * E, T)),
            full((1, T)),
            full((E, H)),
            full((T, H)),
            full((1, H)),
            full((H, C)),
            full((1, C)),
        ],
        out_specs=pl.BlockSpec((blk, C), lambda i: (i, 0)),
        out_shape=jax.ShapeDtypeStruct((B, C), jnp.float32),
    )(trig_feat, rate_feat, W1p, b1.reshape(1, T), W2a, W2b,
      b2.reshape(1, H), W3, b3.reshape(1, C))


def _cast_bf16(table):
    V, E = table.shape
    blk = 20000
    assert V % blk == 0 and blk % 16 == 0

    def body(x_ref, o_ref):
        o_ref[...] = x_ref[...].astype(jnp.bfloat16)

    return pl.pallas_call(
        body,
        grid=(V // blk,),
        in_specs=[pl.BlockSpec((blk, E), lambda i: (i, 0))],
        out_specs=pl.BlockSpec((blk, E), lambda i: (i, 0)),
        out_shape=jax.ShapeDtypeStruct((V, E), jnp.bfloat16),
    )(table)


def kernel(usr_trigram, usr_interacted_rates, trigram_table, rate_table,
           W1, b1, W2, b2, W3, b3):
    B, S, T = usr_trigram.shape
    L = usr_interacted_rates.shape[1]
    E = trigram_table.shape[1]
    H = b2.shape[0]
    C = b3.shape[0]

    trig_bf = _cast_bf16(trigram_table)
    rate_bf = _cast_bf16(rate_table)
    trig_idx_t = usr_trigram.transpose(0, 2, 1).reshape(B * T * S)
    rate_idx = usr_interacted_rates.reshape(B * L)

    trig_feat, rate_feat = _sc_pool(
        trig_idx_t, rate_idx, trig_bf, rate_bf, B, S, T, E, L)
    trig_feat = trig_feat.reshape(B, T * E)

    W1p = (W1.reshape(T, E // 32, 16, 2, T)
           .transpose(0, 1, 3, 2, 4).reshape(T * E, T))
    W2a = (W2[:E].reshape(E // 32, 16, 2, H)
           .transpose(0, 2, 1, 3).reshape(E, H))
    return _mlp(trig_feat, rate_feat, W1p, b1, W2a, W2[E:], b2, W3, b3,
                B, T, E, H, C)


# speedup vs baseline: 1.5425x; 1.5425x over previous
import functools

import jax
import jax.numpy as jnp
from jax import lax
from jax.experimental import pallas as pl
from jax.experimental.pallas import tpu as pltpu
from jax.experimental.pallas import tpu_sc as plsc

_NC = 2
_NS = 16
_NW = _NC * _NS
_LANES = 16


def _sc_pool(trig_idx_t, rate_idx, trig_bf, rate_bf, B, S, T, E, L):
    assert B % (2 * _NW) == 0
    spw = B // _NW
    tph = T // 2
    rph = tph * S
    ch = 80
    assert rph % ch == 0 and ch % 8 == 0
    nch = rph // ch
    ej = E // _LANES
    rchunks = []
    off = 0
    while off < L:
        n = min(128, L - off)
        if L - off > 128:
            n -= n % 8
        rchunks.append((off, n))
        off += n

    mesh = plsc.VectorSubcoreMesh(core_axis_name="c", subcore_axis_name="s")

    @functools.partial(
        pl.kernel,
        out_type=(
            jax.ShapeDtypeStruct((B * T, E), jnp.float32),
            jax.ShapeDtypeStruct((B, E), jnp.float32),
        ),
        mesh=mesh,
        compiler_params=pltpu.CompilerParams(use_tc_tiling_on_sc=False,
                                             needs_layout_passes=False),
        scratch_types=[
            pltpu.VMEM((2, rph), jnp.int32),
            pltpu.VMEM((2, L), jnp.int32),
            pltpu.VMEM((2, rph, E), jnp.bfloat16),
            pltpu.VMEM((2, L, E), jnp.bfloat16),
            pltpu.VMEM((T, E), jnp.float32),
            pltpu.VMEM((1, E), jnp.float32),
            pltpu.SemaphoreType.DMA,
            pltpu.SemaphoreType.DMA,
            pltpu.SemaphoreType.DMA,
            pltpu.SemaphoreType.DMA,
        ],
    )
    def pool(ti_hbm, ri_hbm, tt_hbm, rt_hbm, tout_hbm, rout_hbm,
             idx_v, ridx_v, buf, rbuf, featv, ratev, gsem0, gsem1, rsem0,
             rsem1):
        wid = lax.axis_index("s") * _NC + lax.axis_index("c")
        base_b = wid * spw
        gsems = (gsem0, gsem1)
        rsems = (rsem0, rsem1)
        rps = T * S

        def fire_half(i, half, hb):
            start = (base_b + i) * rps + half * rph
            pltpu.sync_copy(ti_hbm.at[pl.ds(start, rph)], idx_v.at[hb])
            for k in range(nch):
                pltpu.async_copy(
                    tt_hbm.at[idx_v.at[hb, pl.ds(k * ch, ch)]],
                    buf.at[hb, pl.ds(k * ch, ch)], gsems[hb])

        def wait_half(hb):
            pltpu.make_async_copy(
                tt_hbm.at[pl.ds(0, rph)], buf.at[hb], gsems[hb]).wait()

        def fire_rate(i, rb):
            start = (base_b + i) * L
            pltpu.sync_copy(ri_hbm.at[pl.ds(start, L)], ridx_v.at[rb])
            for (o, n) in rchunks:
                pltpu.async_copy(
                    rt_hbm.at[ridx_v.at[rb, pl.ds(o, n)]],
                    rbuf.at[rb, pl.ds(o, n)], rsems[rb])

        def wait_rate(rb):
            pltpu.make_async_copy(
                rt_hbm.at[pl.ds(0, L)], rbuf.at[rb], rsems[rb]).wait()

        def row_terms(ref, *ix):
            terms = []
            for g in range(E // 32):
                packed = ref[(*ix, pl.ds(g * 32, 32))]
                a, b = plsc.unpack(packed,
                                   format=plsc.PackFormat.INTERLEAVED)
                terms.extend((a, b))
            return terms

        def accum_half(half, hb):
            def tbody(tt, c):
                accs = [jnp.zeros((_LANES,), jnp.float32) for _ in range(ej)]
                for s in range(S):
                    terms = row_terms(buf, hb, tt * S + s)
                    for j in range(ej):
                        accs[j] = accs[j] + terms[j]
                for j in range(ej):
                    featv[half * tph + tt, pl.ds(j * _LANES, _LANES)] = (
                        accs[j] * (1.0 / S))
                return c

            lax.fori_loop(0, tph, tbody, 0)

        def accum_rate(rb):
            def rbody(s, accs):
                terms = row_terms(rbuf, rb, s)
                return tuple(accs[j] + terms[j] for j in range(ej))

            raccs = lax.fori_loop(
                0, L, rbody,
                tuple(jnp.zeros((_LANES,), jnp.float32) for _ in range(ej)))
            for j in range(ej):
                ratev[0, pl.ds(j * _LANES, _LANES)] = raccs[j] * (1.0 / L)

        fire_half(0, 0, 0)
        fire_rate(0, 0)

        def pair_body(g, carry):
            for p in range(2):
                i = g * 2 + p
                b = base_b + i
                fire_half(i, 1, 1)
                nxt = jnp.minimum(i + 1, spw - 1)
                fire_rate(nxt, 1 - p)
                wait_half(0)
                accum_half(0, 0)
                fire_half(nxt, 0, 0)
                wait_half(1)
                accum_half(1, 1)
                pltpu.sync_copy(featv, tout_hbm.at[pl.ds(b * T, T)])
                wait_rate(p)
                accum_rate(p)
                pltpu.sync_copy(ratev, rout_hbm.at[pl.ds(b, 1)])
            return carry

        lax.fori_loop(0, spw // 2, pair_body, 0)
        wait_half(0)
        wait_rate(0)

    return pool(trig_idx_t, rate_idx, trig_bf, rate_bf)


def _mlp(trig_feat, rate_feat, W1p, b1, W2a, W2b, b2, W3, b3, B, T, E, H, C):
    blk = 256
    assert B % blk == 0

    def body(tf_ref, rf_ref, w1_ref, b1_ref, w2a_ref, w2b_ref, b2_ref,
             w3_ref, b3_ref, o_ref):
        x = tf_ref[...]
        h1 = jnp.dot(x, w1_ref[...], preferred_element_type=jnp.float32)
        h1 = jnp.maximum(h1 + b1_ref[...], 0.0)
        h2 = (jnp.dot(rf_ref[...], w2a_ref[...],
                      preferred_element_type=jnp.float32)
              + jnp.dot(h1, w2b_ref[...], preferred_element_type=jnp.float32))
        h2 = jnp.maximum(h2 + b2_ref[...], 0.0)
        o_ref[...] = (jnp.dot(h2, w3_ref[...],
                              preferred_element_type=jnp.float32)
                      + b3_ref[...])

    grid = (B // blk,)
    full = lambda shape: pl.BlockSpec(shape, lambda i: (0,) * len(shape))
    return pl.pallas_call(
        body,
        grid=grid,
        in_specs=[
            pl.BlockSpec((blk, T * E), lambda i: (i, 0)),
            pl.BlockSpec((blk, E), lambda i: (i, 0)),
            full((T * E, T)),
            full((1, T)),
            full((E, H)),
            full((T, H)),
            full((1, H)),
            full((H, C)),
            full((1, C)),
        ],
        out_specs=pl.BlockSpec((blk, C), lambda i: (i, 0)),
        out_shape=jax.ShapeDtypeStruct((B, C), jnp.float32),
    )(trig_feat, rate_feat, W1p, b1.reshape(1, T), W2a, W2b,
      b2.reshape(1, H), W3, b3.reshape(1, C))


def _cast_bf16(table):
    return lax.optimization_barrier(table.astype(jnp.bfloat16))


def kernel(usr_trigram, usr_interacted_rates, trigram_table, rate_table,
           W1, b1, W2, b2, W3, b3):
    B, S, T = usr_trigram.shape
    L = usr_interacted_rates.shape[1]
    E = trigram_table.shape[1]
    H = b2.shape[0]
    C = b3.shape[0]

    trig_bf = _cast_bf16(trigram_table)
    rate_bf = _cast_bf16(rate_table)
    trig_idx_t = usr_trigram.transpose(0, 2, 1).reshape(B * T * S)
    rate_idx = usr_interacted_rates.reshape(B * L)

    trig_feat, rate_feat = _sc_pool(
        trig_idx_t, rate_idx, trig_bf, rate_bf, B, S, T, E, L)
    trig_feat = trig_feat.reshape(B, T * E)

    W1p = (W1.reshape(T, E // 32, 16, 2, T)
           .transpose(0, 1, 3, 2, 4).reshape(T * E, T))
    W2a = (W2[:E].reshape(E // 32, 16, 2, H)
           .transpose(0, 2, 1, 3).reshape(E, H))
    return _mlp(trig_feat, rate_feat, W1p, b1, W2a, W2[E:], b2, W3, b3,
                B, T, E, H, C)


# speedup vs baseline: 1.8229x; 1.1817x over previous
import functools

import jax
import jax.numpy as jnp
from jax import lax
from jax.experimental import pallas as pl
from jax.experimental.pallas import tpu as pltpu
from jax.experimental.pallas import tpu_sc as plsc

_NC = 2
_NS = 16
_NW = _NC * _NS
_LANES = 16


def _sc_pool(trig_idx_t, rate_idx, trigram_table, rate_table, B, S, T, E, L):
    assert B % (2 * _NW) == 0
    spw = B // _NW
    tph = T // 2
    rph = tph * S
    ch = 80
    assert rph % ch == 0 and ch % 8 == 0
    nch = rph // ch
    ej = E // _LANES
    rchunks = []
    off = 0
    while off < L:
        n = min(128, L - off)
        if L - off > 128:
            n -= n % 8
        rchunks.append((off, n))
        off += n

    mesh = plsc.VectorSubcoreMesh(core_axis_name="c", subcore_axis_name="s")

    @functools.partial(
        pl.kernel,
        out_type=(
            jax.ShapeDtypeStruct((B * T, E), jnp.float32),
            jax.ShapeDtypeStruct((B, E), jnp.float32),
        ),
        mesh=mesh,
        compiler_params=pltpu.CompilerParams(use_tc_tiling_on_sc=False),
        scratch_types=[
            pltpu.VMEM((2, rph), jnp.int32),
            pltpu.VMEM((2, L), jnp.int32),
            pltpu.VMEM((2, rph, E), jnp.float32),
            pltpu.VMEM((2, L, E), jnp.float32),
            pltpu.VMEM((T, E), jnp.float32),
            pltpu.VMEM((1, E), jnp.float32),
            pltpu.SemaphoreType.DMA,
            pltpu.SemaphoreType.DMA,
            pltpu.SemaphoreType.DMA,
            pltpu.SemaphoreType.DMA,
        ],
    )
    def pool(ti_hbm, ri_hbm, tt_hbm, rt_hbm, tout_hbm, rout_hbm,
             idx_v, ridx_v, buf, rbuf, featv, ratev, gsem0, gsem1, rsem0,
             rsem1):
        wid = lax.axis_index("s") * _NC + lax.axis_index("c")
        base_b = wid * spw
        gsems = (gsem0, gsem1)
        rsems = (rsem0, rsem1)
        rps = T * S

        def fire_half(i, half, hb):
            start = (base_b + i) * rps + half * rph
            pltpu.sync_copy(ti_hbm.at[pl.ds(start, rph)], idx_v.at[hb])
            for k in range(nch):
                pltpu.async_copy(
                    tt_hbm.at[idx_v.at[hb, pl.ds(k * ch, ch)]],
                    buf.at[hb, pl.ds(k * ch, ch)], gsems[hb])

        def wait_half(hb):
            pltpu.make_async_copy(
                tt_hbm.at[pl.ds(0, rph)], buf.at[hb], gsems[hb]).wait()

        def fire_rate(i, rb):
            start = (base_b + i) * L
            pltpu.sync_copy(ri_hbm.at[pl.ds(start, L)], ridx_v.at[rb])
            for (o, n) in rchunks:
                pltpu.async_copy(
                    rt_hbm.at[ridx_v.at[rb, pl.ds(o, n)]],
                    rbuf.at[rb, pl.ds(o, n)], rsems[rb])

        def wait_rate(rb):
            pltpu.make_async_copy(
                rt_hbm.at[pl.ds(0, L)], rbuf.at[rb], rsems[rb]).wait()

        def accum_half(half, hb):
            def tbody(tt, c):
                accs = [jnp.zeros((_LANES,), jnp.float32) for _ in range(ej)]
                for s in range(S):
                    for j in range(ej):
                        accs[j] = accs[j] + buf[hb, tt * S + s,
                                                pl.ds(j * _LANES, _LANES)]
                for j in range(ej):
                    featv[half * tph + tt, pl.ds(j * _LANES, _LANES)] = (
                        accs[j] * (1.0 / S))
                return c

            lax.fori_loop(0, tph, tbody, 0)

        def accum_rate(rb):
            def rbody(s, accs):
                return tuple(
                    accs[j] + rbuf[rb, s, pl.ds(j * _LANES, _LANES)]
                    for j in range(ej))

            raccs = lax.fori_loop(
                0, L, rbody,
                tuple(jnp.zeros((_LANES,), jnp.float32) for _ in range(ej)))
            for j in range(ej):
                ratev[0, pl.ds(j * _LANES, _LANES)] = raccs[j] * (1.0 / L)

        fire_half(0, 0, 0)
        fire_rate(0, 0)

        def pair_body(g, carry):
            for p in range(2):
                i = g * 2 + p
                b = base_b + i
                fire_half(i, 1, 1)
                nxt = jnp.minimum(i + 1, spw - 1)
                fire_rate(nxt, 1 - p)
                wait_half(0)
                accum_half(0, 0)
                fire_half(nxt, 0, 0)
                wait_half(1)
                accum_half(1, 1)
                pltpu.sync_copy(featv, tout_hbm.at[pl.ds(b * T, T)])
                wait_rate(p)
                accum_rate(p)
                pltpu.sync_copy(ratev, rout_hbm.at[pl.ds(b, 1)])
            return carry

        lax.fori_loop(0, spw // 2, pair_body, 0)
        wait_half(0)
        wait_rate(0)

    return pool(trig_idx_t, rate_idx, trigram_table, rate_table)


def _mlp(trig_feat, rate_feat, W1, b1, W2, b2, W3, b3, B, T, E, H, C):
    blk = 256
    assert B % blk == 0

    def body(tf_ref, rf_ref, w1_ref, b1_ref, w2a_ref, w2b_ref, b2_ref,
             w3_ref, b3_ref, o_ref):
        x = tf_ref[...]
        h1 = jnp.dot(x, w1_ref[...], preferred_element_type=jnp.float32)
        h1 = jnp.maximum(h1 + b1_ref[...], 0.0)
        h2 = (jnp.dot(rf_ref[...], w2a_ref[...],
                      preferred_element_type=jnp.float32)
              + jnp.dot(h1, w2b_ref[...], preferred_element_type=jnp.float32))
        h2 = jnp.maximum(h2 + b2_ref[...], 0.0)
        o_ref[...] = (jnp.dot(h2, w3_ref[...],
                              preferred_element_type=jnp.float32)
                      + b3_ref[...])

    grid = (B // blk,)
    full = lambda shape: pl.BlockSpec(shape, lambda i: (0,) * len(shape))
    return pl.pallas_call(
        body,
        grid=grid,
        in_specs=[
            pl.BlockSpec((blk, T * E), lambda i: (i, 0)),
            pl.BlockSpec((blk, E), lambda i: (i, 0)),
            full((T * E, T)),
            full((1, T)),
            full((E, H)),
            full((T, H)),
            full((1, H)),
            full((H, C)),
            full((1, C)),
        ],
        out_specs=pl.BlockSpec((blk, C), lambda i: (i, 0)),
        out_shape=jax.ShapeDtypeStruct((B, C), jnp.float32),
    )(trig_feat, rate_feat, W1, b1.reshape(1, T), W2[:E], W2[E:],
      b2.reshape(1, H), W3, b3.reshape(1, C))


def kernel(usr_trigram, usr_interacted_rates, trigram_table, rate_table,
           W1, b1, W2, b2, W3, b3):
    B, S, T = usr_trigram.shape
    L = usr_interacted_rates.shape[1]
    E = trigram_table.shape[1]
    H = b2.shape[0]
    C = b3.shape[0]

    trig_idx_t = usr_trigram.transpose(0, 2, 1).reshape(B * T * S)
    rate_idx = usr_interacted_rates.reshape(B * L)
    trig_feat, rate_feat = _sc_pool(
        trig_idx_t, rate_idx, trigram_table, rate_table, B, S, T, E, L)
    trig_feat = trig_feat.reshape(B, T * E)
    return _mlp(trig_feat, rate_feat, W1, b1, W2, b2, W3, b3, B, T, E, H, C)


# speedup vs baseline: 2.0039x; 1.0993x over previous
import functools

import jax
import jax.numpy as jnp
from jax import lax
from jax.experimental import pallas as pl
from jax.experimental.pallas import tpu as pltpu
from jax.experimental.pallas import tpu_sc as plsc

_NC = 2
_NS = 16
_NW = _NC * _NS
_LANES = 16


def _sc_trig_pool(trig_idx_t, trigram_table, B, S, T, E):
    assert B % (2 * _NW) == 0
    spw = B // _NW
    tph = T // 2
    rph = tph * S
    ch = 80
    assert rph % ch == 0 and ch % 8 == 0
    nch = rph // ch
    ej = E // _LANES

    mesh = plsc.VectorSubcoreMesh(core_axis_name="c", subcore_axis_name="s")

    @functools.partial(
        pl.kernel,
        out_type=jax.ShapeDtypeStruct((B * T, E), jnp.float32),
        mesh=mesh,
        compiler_params=pltpu.CompilerParams(use_tc_tiling_on_sc=False),
        scratch_types=[
            pltpu.VMEM((2, rph), jnp.int32),
            pltpu.VMEM((2, rph, E), jnp.float32),
            pltpu.VMEM((T, E), jnp.float32),
            pltpu.SemaphoreType.DMA,
            pltpu.SemaphoreType.DMA,
        ],
    )
    def pool(ti_hbm, tt_hbm, tout_hbm, idx_v, buf, featv, gsem0, gsem1):
        wid = lax.axis_index("s") * _NC + lax.axis_index("c")
        base_b = wid * spw
        gsems = (gsem0, gsem1)
        rps = T * S

        def fire_half(i, half, hb):
            start = (base_b + i) * rps + half * rph
            pltpu.sync_copy(ti_hbm.at[pl.ds(start, rph)], idx_v.at[hb])
            for k in range(nch):
                pltpu.async_copy(
                    tt_hbm.at[idx_v.at[hb, pl.ds(k * ch, ch)]],
                    buf.at[hb, pl.ds(k * ch, ch)], gsems[hb])

        def wait_half(hb):
            pltpu.make_async_copy(
                tt_hbm.at[pl.ds(0, rph)], buf.at[hb], gsems[hb]).wait()

        def accum_half(half, hb):
            def tbody(tt, c):
                accs = [jnp.zeros((_LANES,), jnp.float32) for _ in range(ej)]
                for s in range(S):
                    for j in range(ej):
                        accs[j] = accs[j] + buf[hb, tt * S + s,
                                                pl.ds(j * _LANES, _LANES)]
                for j in range(ej):
                    featv[half * tph + tt, pl.ds(j * _LANES, _LANES)] = (
                        accs[j] * (1.0 / S))
                return c

            lax.fori_loop(0, tph, tbody, 0)

        fire_half(0, 0, 0)

        def sample_body(i, carry):
            b = base_b + i
            fire_half(i, 1, 1)
            wait_half(0)
            accum_half(0, 0)
            nxt = jnp.minimum(i + 1, spw - 1)
            fire_half(nxt, 0, 0)
            wait_half(1)
            accum_half(1, 1)
            pltpu.sync_copy(featv, tout_hbm.at[pl.ds(b * T, T)])
            return carry

        lax.fori_loop(0, spw, sample_body, 0)
        wait_half(0)

    return pool(trig_idx_t, trigram_table)


def _sc_rate_pool(rate_idx, rate_table, B, E, L):
    assert B % (2 * _NW) == 0
    spw = B // _NW
    ej = E // _LANES
    rchunks = []
    off = 0
    while off < L:
        n = min(128, L - off)
        if L - off > 128:
            n -= n % 8
        rchunks.append((off, n))
        off += n

    mesh = plsc.VectorSubcoreMesh(core_axis_name="c", subcore_axis_name="s")

    @functools.partial(
        pl.kernel,
        out_type=jax.ShapeDtypeStruct((B, E), jnp.float32),
        mesh=mesh,
        compiler_params=pltpu.CompilerParams(use_tc_tiling_on_sc=False),
        scratch_types=[
            pltpu.VMEM((2, L), jnp.int32),
            pltpu.VMEM((2, L, E), jnp.float32),
            pltpu.VMEM((1, E), jnp.float32),
            pltpu.SemaphoreType.DMA,
            pltpu.SemaphoreType.DMA,
        ],
    )
    def pool(ri_hbm, rt_hbm, rout_hbm, ridx_v, rbuf, ratev, rsem0, rsem1):
        wid = lax.axis_index("s") * _NC + lax.axis_index("c")
        base_b = wid * spw
        rsems = (rsem0, rsem1)

        def fire_rate(i, rb):
            start = (base_b + i) * L
            pltpu.sync_copy(ri_hbm.at[pl.ds(start, L)], ridx_v.at[rb])
            for (o, n) in rchunks:
                pltpu.async_copy(
                    rt_hbm.at[ridx_v.at[rb, pl.ds(o, n)]],
                    rbuf.at[rb, pl.ds(o, n)], rsems[rb])

        def wait_rate(rb):
            pltpu.make_async_copy(
                rt_hbm.at[pl.ds(0, L)], rbuf.at[rb], rsems[rb]).wait()

        def accum_rate(rb):
            def rbody(s, accs):
                return tuple(
                    accs[j] + rbuf[rb, s, pl.ds(j * _LANES, _LANES)]
                    for j in range(ej))

            raccs = lax.fori_loop(
                0, L, rbody,
                tuple(jnp.zeros((_LANES,), jnp.float32) for _ in range(ej)))
            for j in range(ej):
                ratev[0, pl.ds(j * _LANES, _LANES)] = raccs[j] * (1.0 / L)

        fire_rate(0, 0)

        def pair_body(g, carry):
            for p in range(2):
                i = g * 2 + p
                b = base_b + i
                nxt = jnp.minimum(i + 1, spw - 1)
                fire_rate(nxt, 1 - p)
                wait_rate(p)
                accum_rate(p)
                pltpu.sync_copy(ratev, rout_hbm.at[pl.ds(b, 1)])
            return carry

        lax.fori_loop(0, spw // 2, pair_body, 0)
        wait_rate(0)

    return pool(rate_idx, rate_table)


def _mlp(trig_feat, rate_feat, W1, b1, W2, b2, W3, b3, B, T, E, H, C):
    blk = 256
    assert B % blk == 0

    def body(tf_ref, rf_ref, w1_ref, b1_ref, w2a_ref, w2b_ref, b2_ref,
             w3_ref, b3_ref, o_ref):
        x = tf_ref[...]
        h1 = jnp.dot(x, w1_ref[...], preferred_element_type=jnp.float32)
        h1 = jnp.maximum(h1 + b1_ref[...], 0.0)
        h2 = (jnp.dot(rf_ref[...], w2a_ref[...],
                      preferred_element_type=jnp.float32)
              + jnp.dot(h1, w2b_ref[...], preferred_element_type=jnp.float32))
        h2 = jnp.maximum(h2 + b2_ref[...], 0.0)
        o_ref[...] = (jnp.dot(h2, w3_ref[...],
                              preferred_element_type=jnp.float32)
                      + b3_ref[...])

    grid = (B // blk,)
    full = lambda shape: pl.BlockSpec(shape, lambda i: (0,) * len(shape))
    return pl.pallas_call(
        body,
        grid=grid,
        in_specs=[
            pl.BlockSpec((blk, T * E), lambda i: (i, 0)),
            pl.BlockSpec((blk, E), lambda i: (i, 0)),
            full((T * E, T)),
            full((1, T)),
            full((E, H)),
            full((T, H)),
            full((1, H)),
            full((H, C)),
            full((1, C)),
        ],
        out_specs=pl.BlockSpec((blk, C), lambda i: (i, 0)),
        out_shape=jax.ShapeDtypeStruct((B, C), jnp.float32),
    )(trig_feat, rate_feat, W1, b1.reshape(1, T), W2[:E], W2[E:],
      b2.reshape(1, H), W3, b3.reshape(1, C))


def kernel(usr_trigram, usr_interacted_rates, trigram_table, rate_table,
           W1, b1, W2, b2, W3, b3):
    B, S, T = usr_trigram.shape
    L = usr_interacted_rates.shape[1]
    E = trigram_table.shape[1]
    H = b2.shape[0]
    C = b3.shape[0]

    trig_idx_t = usr_trigram.transpose(0, 2, 1).reshape(B * T * S)
    rate_idx = usr_interacted_rates.reshape(B * L)
    trig_feat = _sc_trig_pool(trig_idx_t, trigram_table, B, S, T, E)
    rate_feat = _sc_rate_pool(rate_idx, rate_table, B, E, L)
    trig_feat = trig_feat.reshape(B, T * E)
    return _mlp(trig_feat, rate_feat, W1, b1, W2, b2, W3, b3, B, T, E, H, C)


# speedup vs baseline: 2.0076x; 1.0018x over previous
import functools

import jax
import jax.numpy as jnp
from jax import lax
from jax.experimental import pallas as pl
from jax.experimental.pallas import tpu as pltpu
from jax.experimental.pallas import tpu_sc as plsc

_NC = 2
_NS = 16
_NW = _NC * _NS
_LANES = 16


def _sc_trig_pool(trig_idx_t, trigram_table, B, S, T, E):
    assert B % (2 * _NW) == 0
    spw = B // _NW
    tph = T // 2
    rph = tph * S
    ch = 128
    assert rph % ch == 0 and ch % 8 == 0
    nch = rph // ch
    ej = E // _LANES

    mesh = plsc.VectorSubcoreMesh(core_axis_name="c", subcore_axis_name="s")

    @functools.partial(
        pl.kernel,
        out_type=jax.ShapeDtypeStruct((B * T, E), jnp.float32),
        mesh=mesh,
        compiler_params=pltpu.CompilerParams(use_tc_tiling_on_sc=False),
        scratch_types=[
            pltpu.VMEM((2, rph), jnp.int32),
            pltpu.VMEM((2, rph, E), jnp.float32),
            pltpu.VMEM((T, E), jnp.float32),
            pltpu.SemaphoreType.DMA,
            pltpu.SemaphoreType.DMA,
            pltpu.SemaphoreType.DMA,
            pltpu.SemaphoreType.DMA,
        ],
    )
    def pool(ti_hbm, tt_hbm, tout_hbm, idx_v, buf, featv, gsem0, gsem1,
             isem0, isem1):
        wid = lax.axis_index("s") * _NC + lax.axis_index("c")
        base_b = wid * spw
        gsems = (gsem0, gsem1)
        isems = (isem0, isem1)
        rps = T * S

        def start_idx(i, half, hb):
            start = (base_b + i) * rps + half * rph
            pltpu.async_copy(ti_hbm.at[pl.ds(start, rph)], idx_v.at[hb],
                             isems[hb])

        def fire_half(hb):
            pltpu.make_async_copy(
                ti_hbm.at[pl.ds(0, rph)], idx_v.at[hb], isems[hb]).wait()
            for k in range(nch):
                pltpu.async_copy(
                    tt_hbm.at[idx_v.at[hb, pl.ds(k * ch, ch)]],
                    buf.at[hb, pl.ds(k * ch, ch)], gsems[hb])

        def wait_half(hb):
            pltpu.make_async_copy(
                tt_hbm.at[pl.ds(0, rph)], buf.at[hb], gsems[hb]).wait()

        def accum_half(half, hb):
            def tbody(tt, c):
                accs = [jnp.zeros((_LANES,), jnp.float32) for _ in range(ej)]
                for s in range(S):
                    for j in range(ej):
                        accs[j] = accs[j] + buf[hb, tt * S + s,
                                                pl.ds(j * _LANES, _LANES)]
                for j in range(ej):
                    featv[half * tph + tt, pl.ds(j * _LANES, _LANES)] = (
                        accs[j] * (1.0 / S))
                return c

            lax.fori_loop(0, tph, tbody, 0)

        start_idx(0, 0, 0)
        fire_half(0)
        start_idx(0, 1, 1)

        def sample_body(i, carry):
            b = base_b + i
            nxt = jnp.minimum(i + 1, spw - 1)
            fire_half(1)
            wait_half(0)
            accum_half(0, 0)
            start_idx(nxt, 0, 0)
            fire_half(0)
            wait_half(1)
            accum_half(1, 1)
            start_idx(nxt, 1, 1)
            pltpu.sync_copy(featv, tout_hbm.at[pl.ds(b * T, T)])
            return carry

        lax.fori_loop(0, spw, sample_body, 0)
        wait_half(0)
        pltpu.make_async_copy(
            ti_hbm.at[pl.ds(0, rph)], idx_v.at[1], isem1).wait()

    return pool(trig_idx_t, trigram_table)


def _sc_rate_pool(rate_idx, rate_table, B, E, L):
    assert B % (2 * _NW) == 0
    spw = B // _NW
    ej = E // _LANES
    rchunks = []
    off = 0
    while off < L:
        n = min(128, L - off)
        if L - off > 128:
            n -= n % 8
        rchunks.append((off, n))
        off += n

    mesh = plsc.VectorSubcoreMesh(core_axis_name="c", subcore_axis_name="s")

    @functools.partial(
        pl.kernel,
        out_type=jax.ShapeDtypeStruct((B, E), jnp.float32),
        mesh=mesh,
        compiler_params=pltpu.CompilerParams(use_tc_tiling_on_sc=False),
        scratch_types=[
            pltpu.VMEM((2, L), jnp.int32),
            pltpu.VMEM((2, L, E), jnp.float32),
            pltpu.VMEM((1, E), jnp.float32),
            pltpu.SemaphoreType.DMA,
            pltpu.SemaphoreType.DMA,
        ],
    )
    def pool(ri_hbm, rt_hbm, rout_hbm, ridx_v, rbuf, ratev, rsem0, rsem1):
        wid = lax.axis_index("s") * _NC + lax.axis_index("c")
        base_b = wid * spw
        rsems = (rsem0, rsem1)

        def fire_rate(i, rb):
            start = (base_b + i) * L
            pltpu.sync_copy(ri_hbm.at[pl.ds(start, L)], ridx_v.at[rb])
            for (o, n) in rchunks:
                pltpu.async_copy(
                    rt_hbm.at[ridx_v.at[rb, pl.ds(o, n)]],
                    rbuf.at[rb, pl.ds(o, n)], rsems[rb])

        def wait_rate(rb):
            pltpu.make_async_copy(
                rt_hbm.at[pl.ds(0, L)], rbuf.at[rb], rsems[rb]).wait()

        def accum_rate(rb):
            def rbody(s, accs):
                return tuple(
                    accs[j] + rbuf[rb, s, pl.ds(j * _LANES, _LANES)]
                    for j in range(ej))

            raccs = lax.fori_loop(
                0, L, rbody,
                tuple(jnp.zeros((_LANES,), jnp.float32) for _ in range(ej)))
            for j in range(ej):
                ratev[0, pl.ds(j * _LANES, _LANES)] = raccs[j] * (1.0 / L)

        fire_rate(0, 0)

        def pair_body(g, carry):
            for p in range(2):
                i = g * 2 + p
                b = base_b + i
                nxt = jnp.minimum(i + 1, spw - 1)
                fire_rate(nxt, 1 - p)
                wait_rate(p)
                accum_rate(p)
                pltpu.sync_copy(ratev, rout_hbm.at[pl.ds(b, 1)])
            return carry

        lax.fori_loop(0, spw // 2, pair_body, 0)
        wait_rate(0)

    return pool(rate_idx, rate_table)


def _mlp(trig_feat, rate_feat, W1, b1, W2, b2, W3, b3, B, T, E, H, C):
    blk = 256
    assert B % blk == 0

    def body(tf_ref, rf_ref, w1_ref, b1_ref, w2a_ref, w2b_ref, b2_ref,
             w3_ref, b3_ref, o_ref):
        x = tf_ref[...]
        h1 = jnp.dot(x, w1_ref[...], preferred_element_type=jnp.float32)
        h1 = jnp.maximum(h1 + b1_ref[...], 0.0)
        h2 = (jnp.dot(rf_ref[...], w2a_ref[...],
                      preferred_element_type=jnp.float32)
              + jnp.dot(h1, w2b_ref[...], preferred_element_type=jnp.float32))
        h2 = jnp.maximum(h2 + b2_ref[...], 0.0)
        o_ref[...] = (jnp.dot(h2, w3_ref[...],
                              preferred_element_type=jnp.float32)
                      + b3_ref[...])

    grid = (B // blk,)
    full = lambda shape: pl.BlockSpec(shape, lambda i: (0,) * len(shape))
    return pl.pallas_call(
        body,
        grid=grid,
        in_specs=[
            pl.BlockSpec((blk, T * E), lambda i: (i, 0)),
            pl.BlockSpec((blk, E), lambda i: (i, 0)),
            full((T * E, T)),
            full((1, T)),
            full((E, H)),
            full((T, H)),
            full((1, H)),
            full((H, C)),
            full((1, C)),
        ],
        out_specs=pl.BlockSpec((blk, C), lambda i: (i, 0)),
        out_shape=jax.ShapeDtypeStruct((B, C), jnp.float32),
    )(trig_feat, rate_feat, W1, b1.reshape(1, T), W2[:E], W2[E:],
      b2.reshape(1, H), W3, b3.reshape(1, C))


def kernel(usr_trigram, usr_interacted_rates, trigram_table, rate_table,
           W1, b1, W2, b2, W3, b3):
    B, S, T = usr_trigram.shape
    L = usr_interacted_rates.shape[1]
    E = trigram_table.shape[1]
    H = b2.shape[0]
    C = b3.shape[0]

    trig_idx_t = usr_trigram.transpose(0, 2, 1).reshape(B * T * S)
    rate_idx = usr_interacted_rates.reshape(B * L)
    trig_feat = _sc_trig_pool(trig_idx_t, trigram_table, B, S, T, E)
    rate_feat = _sc_rate_pool(rate_idx, rate_table, B, E, L)
    trig_feat = trig_feat.reshape(B, T * E)
    return _mlp(trig_feat, rate_feat, W1, b1, W2, b2, W3, b3, B, T, E, H, C)
